# baseline (speedup 1.0000x reference)
import jax
import jax.numpy as jnp
import numpy as np
from jax.experimental import pallas as pl

_NUM_SPECIES = 4
_NUM_PAIRS = 10
_RCR = 5.2
_RCA = 3.5
_ETA_R = 16.0
_ETA_A = 8.0
_ZETA_LOG2 = 5


def _pow_zeta(x):
    for _ in range(_ZETA_LOG2):
        x = x * x
    return x


def _mol_body(sp_ref, xyz_ref, out_ref):
    A = sp_ref.shape[-1]
    sp = sp_ref[0, 0, :]
    xyz = xyz_ref[0]

    eye = (jax.lax.broadcasted_iota(jnp.int32, (A, A), 0)
           == jax.lax.broadcasted_iota(jnp.int32, (A, A), 1))
    jlk = (jax.lax.broadcasted_iota(jnp.int32, (A, A), 0)
           < jax.lax.broadcasted_iota(jnp.int32, (A, A), 1))

    x, y, z = xyz[:, 0], xyz[:, 1], xyz[:, 2]
    dx = x[None, :] - x[:, None]
    dy = y[None, :] - y[:, None]
    dz = z[None, :] - z[:, None]
    d2 = dx * dx + dy * dy + dz * dz
    dist = jnp.sqrt(jnp.where(eye, 1.0, d2))
    offdiag = ~eye

    fc_r = 0.5 * jnp.cos(jnp.pi * dist / _RCR) + 0.5
    mask_r = offdiag & (dist <= _RCR)
    shf_r = (0.9 + (_RCR - 0.9) / 16.0
             * jax.lax.broadcasted_iota(jnp.int32, (1, 1, 16), 2
                                        ).astype(jnp.float32))
    fc_rm = jnp.where(mask_r, fc_r, 0.0)
    rad = (0.25 * jnp.exp(-_ETA_R * (dist[:, :, None] - shf_r) ** 2)
           * fc_rm[:, :, None])
    oh_s = (sp[:, None]
            == jax.lax.broadcasted_iota(jnp.int32, (1, _NUM_SPECIES), 1)
            ).astype(jnp.float32)
    radial = jnp.einsum('ijr,js->isr', rad, oh_s,
                        preferred_element_type=jnp.float32)

    inner = (dx[:, :, None] * dx[:, None, :]
             + dy[:, :, None] * dy[:, None, :]
             + dz[:, :, None] * dz[:, None, :])
    denom = jnp.maximum(dist[:, :, None] * dist[:, None, :], 1e-10)
    c95 = 0.95 * jnp.clip(inner / denom, -1.0, 1.0)
    sin_t = jnp.sqrt(1.0 - c95 * c95)

    fc_a = 0.5 * jnp.cos(jnp.pi * dist / _RCA) + 0.5
    mask_a = offdiag & (dist <= _RCA)
    fc_am = jnp.where(mask_a, fc_a, 0.0)
    jlkf = jnp.where(jlk, 1.0, 0.0)
    gate = (fc_am[:, :, None] * fc_am[:, None, :]) * jlkf[None, :, :]

    shf_z = (np.float32(np.pi) / 8.0
             * (jax.lax.broadcasted_iota(jnp.int32, (1, 1, 1, 8), 3
                                         ).astype(jnp.float32) + 0.5))
    cos_z = jnp.cos(shf_z)
    sin_z = jnp.sin(shf_z)
    f1 = _pow_zeta(0.5 * (1.0 + c95[..., None] * cos_z
                          + sin_t[..., None] * sin_z))
    avg = 0.5 * (dist[:, :, None] + dist[:, None, :])
    shf_a = (0.9 + (_RCA - 0.9) / 4.0
             * jax.lax.broadcasted_iota(jnp.int32, (1, 1, 1, 4), 3
                                        ).astype(jnp.float32))
    f2 = jnp.exp(-_ETA_A * (avg[..., None] - shf_a) ** 2)
    f2g = f2 * (2.0 * gate)[..., None]
    term = jnp.concatenate(
        [f2g[..., a:a + 1] * f1 for a in range(4)], axis=-1)
    term = term.reshape(A, A * A, 32)

    pair_ab = [(0, 0), (0, 1), (0, 2), (0, 3), (1, 1),
               (1, 2), (1, 3), (2, 2), (2, 3), (3, 3)]
    oh_cols = [oh_s[:, a] for a in range(_NUM_SPECIES)]
    oh_p_list = []
    for (a, b) in pair_ab:
        outer = oh_cols[a][:, None] * oh_cols[b][None, :]
        if a != b:
            outer = outer + oh_cols[b][:, None] * oh_cols[a][None, :]
        oh_p_list.append(outer.reshape(A, A, 1))
    oh_p = jnp.concatenate(oh_p_list, axis=2).reshape(A * A, _NUM_PAIRS)
    angular = jnp.einsum('ixt,xp->ipt', term, oh_p,
                         preferred_element_type=jnp.float32)

    out_ref[0] = jnp.concatenate(
        [radial.reshape(A, _NUM_SPECIES * 16),
         angular.reshape(A, _NUM_PAIRS * 32)], axis=1)


def kernel(species, coordinates):
    N, A = species.shape
    sp32 = species.astype(jnp.int32).reshape(N, 1, A)
    aev = pl.pallas_call(
        _mol_body,
        grid=(N,),
        in_specs=[
            pl.BlockSpec((1, 1, A), lambda n: (n, 0, 0)),
            pl.BlockSpec((1, A, 3), lambda n: (n, 0, 0)),
        ],
        out_specs=pl.BlockSpec((1, A, 384), lambda n: (n, 0, 0)),
        out_shape=jax.ShapeDtypeStruct((N, A, 384), jnp.float32),
    )(sp32, coordinates)
    return (species, aev)


# speedup vs baseline: 6.9429x; 6.9429x over previous
import jax
import jax.numpy as jnp
import numpy as np
from jax.experimental import pallas as pl

_NUM_SPECIES = 4
_NUM_PAIRS = 10
_RCR = 5.2
_RCA = 3.5
_ETA_R = 16.0
_ETA_A = 8.0
_ZETA_LOG2 = 5
_PAIR_AB = [(0, 0), (0, 1), (0, 2), (0, 3), (1, 1),
            (1, 2), (1, 3), (2, 2), (2, 3), (3, 3)]


def _pow_zeta(x):
    for _ in range(_ZETA_LOG2):
        x = x * x
    return x


def _iota(shape, dim):
    return jax.lax.broadcasted_iota(jnp.int32, shape, dim)


def _mol_body(sp_ref, xyz_ref, out_ref):
    A = sp_ref.shape[-1]
    JK = A * A
    HI = jax.lax.Precision.HIGHEST
    sp = sp_ref[0, 0, :]
    xyz = xyz_ref[0]

    eye = _iota((A, A), 0) == _iota((A, A), 1)

    x, y, z = xyz[:, 0], xyz[:, 1], xyz[:, 2]
    dx = x[None, :] - x[:, None]
    dy = y[None, :] - y[:, None]
    dz = z[None, :] - z[:, None]
    d2 = dx * dx + dy * dy + dz * dz
    dist = jnp.sqrt(jnp.where(eye, 1.0, d2))
    offdiag = ~eye

    fc_r = 0.5 * jnp.cos(jnp.pi * dist / _RCR) + 0.5
    mask_r = offdiag & (dist <= _RCR)
    shf_r = (0.9 + (_RCR - 0.9) / 16.0
             * _iota((1, 1, 16), 2).astype(jnp.float32))
    fc_rm = jnp.where(mask_r, fc_r, 0.0)
    rad = (0.25 * jnp.exp(-_ETA_R * (dist[:, :, None] - shf_r) ** 2)
           * fc_rm[:, :, None])
    oh_s = (sp[:, None]
            == _iota((1, _NUM_SPECIES), 1)).astype(jnp.float32)
    radial = jnp.einsum('ijr,js->isr', rad, oh_s,
                        preferred_element_type=jnp.float32)

    fc_a = 0.5 * jnp.cos(jnp.pi * dist / _RCA) + 0.5
    mask_a = offdiag & (dist <= _RCA)
    fc_am = jnp.where(mask_a, fc_a, 0.0)

    E1 = (_iota((A, JK), 1) // A == _iota((A, JK), 0)).astype(jnp.float32)
    E2 = (_iota((A, JK), 1) % A == _iota((A, JK), 0)).astype(jnp.float32)
    d_1 = jnp.dot(dist, E1, precision=HI)
    d_2 = jnp.dot(dist, E2, precision=HI)
    dx1 = jnp.dot(dx, E1, precision=HI)
    dx2 = jnp.dot(dx, E2, precision=HI)
    dy1 = jnp.dot(dy, E1, precision=HI)
    dy2 = jnp.dot(dy, E2, precision=HI)
    dz1 = jnp.dot(dz, E1, precision=HI)
    dz2 = jnp.dot(dz, E2, precision=HI)
    fc1 = jnp.dot(fc_am, E1, precision=HI)
    fc2 = jnp.dot(fc_am, E2, precision=HI)

    inner = dx1 * dx2 + dy1 * dy2 + dz1 * dz2
    denom = jnp.maximum(d_1 * d_2, 1e-10)
    c95 = 0.95 * jnp.clip(inner / denom, -1.0, 1.0)
    sin_t = jnp.sqrt(1.0 - c95 * c95)
    avg = 0.5 * (d_1 + d_2)
    lane = _iota((1, JK), 1)
    jlk = ((lane // A) < (lane % A)).astype(jnp.float32)
    gate2 = 2.0 * fc1 * fc2 * jlk

    spf = sp.astype(jnp.float32)[:, None]
    E1T = (_iota((JK, A), 0) // A == _iota((JK, A), 1)).astype(jnp.float32)
    E2T = (_iota((JK, A), 0) % A == _iota((JK, A), 1)).astype(jnp.float32)
    s1 = jnp.dot(E1T, spf)
    s2 = jnp.dot(E2T, spf)
    cols = []
    for (a, b) in _PAIR_AB:
        w = jnp.where((s1 == float(a)) & (s2 == float(b)), 1.0, 0.0)
        if a != b:
            w = w + jnp.where((s1 == float(b)) & (s2 == float(a)), 1.0, 0.0)
        cols.append(w)
    oh_p = jnp.concatenate(cols, axis=1)

    f1s = []
    for zi in range(8):
        shz = np.pi * (zi + 0.5) / 8.0
        czv, szv = float(np.cos(shz)), float(np.sin(shz))
        f1s.append(_pow_zeta(0.5 * (1.0 + c95 * czv + sin_t * szv)))
    outs = []
    for a in range(4):
        sha = 0.9 + (_RCA - 0.9) / 4.0 * a
        f2g = jnp.exp(-_ETA_A * (avg - sha) ** 2) * gate2
        for zi in range(8):
            outs.append(jnp.dot(f2g * f1s[zi], oh_p,
                                preferred_element_type=jnp.float32))
    angt = jnp.concatenate(outs, axis=1)
    r_i = _iota((320, 320), 0)
    c_i = _iota((320, 320), 1)
    P = ((r_i % 10) * 32 + r_i // 10 == c_i).astype(jnp.float32)
    ang = jnp.dot(angt, P, precision=HI)

    out_ref[0] = jnp.concatenate(
        [radial.reshape(A, _NUM_SPECIES * 16), ang], axis=1)


def kernel(species, coordinates):
    N, A = species.shape
    sp32 = species.astype(jnp.int32).reshape(N, 1, A)
    aev = pl.pallas_call(
        _mol_body,
        grid=(N,),
        in_specs=[
            pl.BlockSpec((1, 1, A), lambda n: (n, 0, 0)),
            pl.BlockSpec((1, A, 3), lambda n: (n, 0, 0)),
        ],
        out_specs=pl.BlockSpec((1, A, 384), lambda n: (n, 0, 0)),
        out_shape=jax.ShapeDtypeStruct((N, A, 384), jnp.float32),
    )(sp32, coordinates)
    return (species, aev)
